# baseline (speedup 1.0000x reference)
import functools

import jax
import jax.numpy as jnp
from jax import lax
from jax.experimental import pallas as pl
from jax.experimental.pallas import tpu as pltpu
from jax.experimental.pallas import tpu_sc as plsc

E = 160000
D = 256
G = 64
BE = 16000
NB = E // BE

_NEG_INF = float("-inf")


def _pass1_body(seg_ref, feats_ref, wt_ref, b_ref, e2_ref, h_ref, c_ref,
                m_scr, s_scr, acc_scr):
    i = pl.program_id(0)

    @pl.when(i == 0)
    def _init():
        m_scr[...] = jnp.full_like(m_scr, _NEG_INF)
        s_scr[...] = jnp.zeros_like(s_scr)
        acc_scr[...] = jnp.zeros_like(acc_scr)

    feats = feats_ref[...]
    e2 = jax.lax.dot_general(wt_ref[...], feats, (((1,), (1,)), ((), ())),
                             preferred_element_type=jnp.float32)
    e2 = e2 + b_ref[0]
    e2 = jnp.where(e2 >= 0, e2, 0.01 * e2)
    e2_ref[0] = e2

    seg = seg_ref[0]
    gids = jax.lax.broadcasted_iota(jnp.int32, (G, BE), 0)
    mask = seg == gids
    e2m = jnp.where(mask, e2, _NEG_INF)
    bm = jnp.max(e2m, axis=1, keepdims=True)

    m_old = m_scr[...]
    m_new = jnp.maximum(m_old, bm)
    m_safe = jnp.where(m_new == _NEG_INF, 0.0, m_new)
    factor = jnp.where(m_new == _NEG_INF, 0.0, jnp.exp(m_old - m_new))

    m_e = jnp.sum(jnp.where(mask, m_safe, 0.0), axis=0, keepdims=True)
    p = jnp.exp(e2 - m_e)
    p2 = jnp.where(mask, p, 0.0)

    m_scr[...] = m_new
    s_scr[...] = s_scr[...] * factor + jnp.sum(p2, axis=1, keepdims=True)
    pf = jax.lax.dot_general(p2, feats, (((1,), (0,)), ((), ())),
                             preferred_element_type=jnp.float32)
    acc_scr[...] = acc_scr[...] * factor + pf

    @pl.when(i == NB - 1)
    def _finish():
        s = s_scr[...]
        m = m_scr[...]
        c_ref[...] = jnp.where(s > 0, m + jnp.log(s), 0.0)
        h_ref[...] = jnp.where(s > 0, acc_scr[...] / s, 0.0)


_NC = 2
_NS = 16
_NW = _NC * _NS
_EPAD = 160256
_CHUNK = _EPAD // _NW
_NVEC = _CHUNK // 16


def _pass2_sc(seg_hbm, e2_hbm, c_hbm, w_hbm, idx_v, e2_v, w_v, c_v):
    wid = lax.axis_index("s") * _NC + lax.axis_index("c")
    base = wid * _CHUNK
    pltpu.sync_copy(seg_hbm.at[pl.ds(base, _CHUNK)], idx_v)
    pltpu.sync_copy(e2_hbm.at[pl.ds(base, _CHUNK)], e2_v)
    pltpu.sync_copy(c_hbm, c_v)

    def body(j, _):
        o = j * 16
        idx16 = idx_v[pl.ds(o, 16)]
        cv = plsc.load_gather(c_v, [idx16])
        w_v[pl.ds(o, 16)] = jnp.exp(e2_v[pl.ds(o, 16)] - cv)
        return _

    lax.fori_loop(0, _NVEC, body, None)
    pltpu.sync_copy(w_v, w_hbm.at[pl.ds(base, _CHUNK)])


@functools.partial(jax.jit, static_argnames=("interpret",))
def _run(edge_feats, segment_ids, W, b, interpret=False):
    seg3 = segment_ids.astype(jnp.int32).reshape(NB, 1, BE)
    wt = W.reshape(1, D)

    e23, h, c = pl.pallas_call(
        _pass1_body,
        grid=(NB,),
        in_specs=[
            pl.BlockSpec((1, 1, BE), lambda i: (i, 0, 0)),
            pl.BlockSpec((BE, D), lambda i: (i, 0)),
            pl.BlockSpec((1, D), lambda i: (0, 0)),
            pl.BlockSpec(memory_space=pltpu.SMEM),
        ],
        out_specs=[
            pl.BlockSpec((1, 1, BE), lambda i: (i, 0, 0)),
            pl.BlockSpec((G, D), lambda i: (0, 0)),
            pl.BlockSpec((G, 1), lambda i: (0, 0)),
        ],
        out_shape=[
            jax.ShapeDtypeStruct((NB, 1, BE), jnp.float32),
            jax.ShapeDtypeStruct((G, D), jnp.float32),
            jax.ShapeDtypeStruct((G, 1), jnp.float32),
        ],
        scratch_shapes=[
            pltpu.VMEM((G, 1), jnp.float32),
            pltpu.VMEM((G, 1), jnp.float32),
            pltpu.VMEM((G, D), jnp.float32),
        ],
        interpret=interpret,
    )(seg3, edge_feats, wt, b)

    seg_flat = jnp.concatenate(
        [segment_ids.astype(jnp.int32), jnp.zeros((_EPAD - E,), jnp.int32)])
    e2_flat = jnp.concatenate(
        [e23.reshape(E), jnp.zeros((_EPAD - E,), jnp.float32)])

    sc_kernel = functools.partial(
        pl.kernel,
        mesh=plsc.VectorSubcoreMesh(core_axis_name="c", subcore_axis_name="s"),
        compiler_params=pltpu.CompilerParams(needs_layout_passes=False),
        out_type=jax.ShapeDtypeStruct((_EPAD,), jnp.float32),
        scratch_types=[
            pltpu.VMEM((_CHUNK,), jnp.int32),
            pltpu.VMEM((_CHUNK,), jnp.float32),
            pltpu.VMEM((_CHUNK,), jnp.float32),
            pltpu.VMEM((128,), jnp.float32),
        ],
    )(_pass2_sc)
    w_flat = sc_kernel(seg_flat, e2_flat, jnp.concatenate([c.reshape(G), jnp.zeros((128 - G,), jnp.float32)]))

    return h, w_flat[:E].reshape(E, 1)


def kernel(edge_feats, segment_ids, W, b, num_graphs):
    del num_graphs
    return _run(edge_feats, segment_ids, W, b)


# speedup vs baseline: 1.0312x; 1.0312x over previous
import functools

import jax
import jax.numpy as jnp
from jax import lax
from jax.experimental import pallas as pl
from jax.experimental.pallas import tpu as pltpu
from jax.experimental.pallas import tpu_sc as plsc

E = 160000
D = 256
G = 64
BE = 16000
NB = E // BE

_NEG_INF = float("-inf")


def _pass1_body(seg_ref, feats_ref, wt_ref, b_ref, e2_ref, h_ref, c_ref,
                m_scr, s_scr, acc_scr):
    i = pl.program_id(0)

    @pl.when(i == 0)
    def _init():
        m_scr[...] = jnp.full_like(m_scr, _NEG_INF)
        s_scr[...] = jnp.zeros_like(s_scr)
        acc_scr[...] = jnp.zeros_like(acc_scr)

    feats = feats_ref[...]
    e2 = jax.lax.dot_general(wt_ref[...], feats, (((1,), (1,)), ((), ())),
                             preferred_element_type=jnp.float32)
    e2 = e2 + b_ref[0]
    e2 = jnp.where(e2 >= 0, e2, 0.01 * e2)
    e2_ref[pl.ds(i * BE, BE)] = e2.reshape(BE)

    seg = seg_ref[0]
    gids = jax.lax.broadcasted_iota(jnp.int32, (G, BE), 0)
    mask = seg == gids
    e2m = jnp.where(mask, e2, _NEG_INF)
    bm = jnp.max(e2m, axis=1, keepdims=True)

    m_old = m_scr[...]
    m_new = jnp.maximum(m_old, bm)
    m_safe = jnp.where(m_new == _NEG_INF, 0.0, m_new)
    factor = jnp.where(m_new == _NEG_INF, 0.0, jnp.exp(m_old - m_new))

    m_e = jnp.sum(jnp.where(mask, m_safe, 0.0), axis=0, keepdims=True)
    p = jnp.exp(e2 - m_e)
    p2 = jnp.where(mask, p, 0.0)

    m_scr[...] = m_new
    s_scr[...] = s_scr[...] * factor + jnp.sum(p2, axis=1, keepdims=True)
    pf = jax.lax.dot_general(p2, feats, (((1,), (0,)), ((), ())),
                             preferred_element_type=jnp.float32)
    acc_scr[...] = acc_scr[...] * factor + pf

    @pl.when(i == NB - 1)
    def _finish():
        s = s_scr[...]
        m = m_scr[...]
        c_ref[...] = jnp.where(s > 0, m + jnp.log(s), 0.0)
        h_ref[...] = jnp.where(s > 0, acc_scr[...] / s, 0.0)


_NC = 2
_NS = 16
_NW = _NC * _NS
_CHUNK = E // _NW
_NVEC = -(-_CHUNK // 16)


def _pass2_sc(seg_hbm, e2_hbm, c_hbm, w_hbm, idx_v, e2_v, w_v, c_v):
    wid = lax.axis_index("s") * _NC + lax.axis_index("c")
    base = wid * _CHUNK
    pltpu.sync_copy(seg_hbm.at[pl.ds(base, _CHUNK)], idx_v)
    pltpu.sync_copy(e2_hbm.at[pl.ds(base, _CHUNK)], e2_v)
    pltpu.sync_copy(c_hbm, c_v)

    def body(j, _):
        o = jnp.minimum(j * 16, _CHUNK - 16)
        idx16 = idx_v[pl.ds(o, 16)]
        cv = plsc.load_gather(c_v, [idx16])
        w_v[pl.ds(o, 16)] = jnp.exp(e2_v[pl.ds(o, 16)] - cv)
        return _

    lax.fori_loop(0, _NVEC, body, None, unroll=8)
    pltpu.sync_copy(w_v, w_hbm.at[pl.ds(base, _CHUNK)])


@functools.partial(jax.jit, static_argnames=("interpret",))
def _run(edge_feats, segment_ids, W, b, interpret=False):
    seg3 = segment_ids.astype(jnp.int32).reshape(NB, 1, BE)
    wt = W.reshape(1, D)

    e2_flat, h, c = pl.pallas_call(
        _pass1_body,
        grid=(NB,),
        in_specs=[
            pl.BlockSpec((1, 1, BE), lambda i: (i, 0, 0)),
            pl.BlockSpec((BE, D), lambda i: (i, 0)),
            pl.BlockSpec((1, D), lambda i: (0, 0)),
            pl.BlockSpec(memory_space=pltpu.SMEM),
        ],
        out_specs=[
            pl.BlockSpec((E,), lambda i: (0,)),
            pl.BlockSpec((G, D), lambda i: (0, 0)),
            pl.BlockSpec((G, 1), lambda i: (0, 0)),
        ],
        out_shape=[
            jax.ShapeDtypeStruct((E,), jnp.float32),
            jax.ShapeDtypeStruct((G, D), jnp.float32),
            jax.ShapeDtypeStruct((G, 1), jnp.float32),
        ],
        scratch_shapes=[
            pltpu.VMEM((G, 1), jnp.float32),
            pltpu.VMEM((G, 1), jnp.float32),
            pltpu.VMEM((G, D), jnp.float32),
        ],
        interpret=interpret,
    )(seg3, edge_feats, wt, b)

    seg_flat = segment_ids.astype(jnp.int32)
    c_pad = jnp.concatenate([c.reshape(G), jnp.zeros((128 - G,), jnp.float32)])

    sc_kernel = functools.partial(
        pl.kernel,
        mesh=plsc.VectorSubcoreMesh(core_axis_name="c", subcore_axis_name="s"),
        compiler_params=pltpu.CompilerParams(needs_layout_passes=False),
        out_type=jax.ShapeDtypeStruct((E,), jnp.float32),
        scratch_types=[
            pltpu.VMEM((_CHUNK,), jnp.int32),
            pltpu.VMEM((_CHUNK,), jnp.float32),
            pltpu.VMEM((_CHUNK,), jnp.float32),
            pltpu.VMEM((128,), jnp.float32),
        ],
    )(_pass2_sc)
    w_flat = sc_kernel(seg_flat, e2_flat, c_pad)

    return h, w_flat.reshape(E, 1)


def kernel(edge_feats, segment_ids, W, b, num_graphs):
    del num_graphs
    return _run(edge_feats, segment_ids, W, b)


# speedup vs baseline: 1.0765x; 1.0439x over previous
import functools

import jax
import jax.numpy as jnp
from jax import lax
from jax.experimental import pallas as pl
from jax.experimental.pallas import tpu as pltpu
from jax.experimental.pallas import tpu_sc as plsc

E = 160000
D = 256
G = 64
BE = 16000
NB = E // BE

_NEG_INF = float("-inf")


def _pass1_body(seg_ref, feats_ref, wt_ref, b_ref, e2_ref, h_ref, c_ref,
                m_scr, s_scr, acc_scr):
    i = pl.program_id(0)

    @pl.when(i == 0)
    def _init():
        m_scr[...] = jnp.full_like(m_scr, _NEG_INF)
        s_scr[...] = jnp.zeros_like(s_scr)
        acc_scr[...] = jnp.zeros_like(acc_scr)

    feats = feats_ref[...]
    e2 = jax.lax.dot_general(wt_ref[...], feats, (((1,), (1,)), ((), ())),
                             preferred_element_type=jnp.float32)
    e2 = e2 + b_ref[0]
    e2 = jnp.where(e2 >= 0, e2, 0.01 * e2)
    e2_ref[pl.ds(i * BE, BE)] = e2.reshape(BE)

    seg = seg_ref[0]
    gids = jax.lax.broadcasted_iota(jnp.int32, (G, BE), 0)
    mask = seg == gids
    e2m = jnp.where(mask, e2, _NEG_INF)
    bm = jnp.max(e2m, axis=1, keepdims=True)

    m_old = m_scr[...]
    m_new = jnp.maximum(m_old, bm)
    m_safe = jnp.where(m_new == _NEG_INF, 0.0, m_new)
    factor = jnp.where(m_new == _NEG_INF, 0.0, jnp.exp(m_old - m_new))

    m_e = jnp.sum(jnp.where(mask, m_safe, 0.0), axis=0, keepdims=True)
    p = jnp.exp(e2 - m_e)
    p2 = jnp.where(mask, p, 0.0)

    m_scr[...] = m_new
    s_scr[...] = s_scr[...] * factor + jnp.sum(p2, axis=1, keepdims=True)
    pf = jax.lax.dot_general(p2, feats, (((1,), (0,)), ((), ())),
                             preferred_element_type=jnp.float32)
    acc_scr[...] = acc_scr[...] * factor + pf

    @pl.when(i == NB - 1)
    def _finish():
        s = s_scr[...]
        m = m_scr[...]
        c_ref[...] = jnp.where(s > 0, m + jnp.log(s), 0.0)
        h_ref[...] = jnp.where(s > 0, acc_scr[...] / s, 0.0)


_NC = 2
_NS = 16
_NW = _NC * _NS
_CHUNK = E // _NW
_NVEC = -(-_CHUNK // 16)


def _pass2_sc(seg_hbm, e2_hbm, c_hbm, w_hbm, idx_v, e2_v, w_v, c_v):
    wid = lax.axis_index("s") * _NC + lax.axis_index("c")
    base = wid * _CHUNK
    pltpu.sync_copy(seg_hbm.at[pl.ds(base, _CHUNK)], idx_v)
    pltpu.sync_copy(e2_hbm.at[pl.ds(base, _CHUNK)], e2_v)
    pltpu.sync_copy(c_hbm, c_v)

    @plsc.parallel_loop(0, _NVEC, unroll=8)
    def _body(j):
        o = jnp.minimum(j * 16, _CHUNK - 16)
        idx16 = idx_v[pl.ds(o, 16)]
        cv = plsc.load_gather(c_v, [idx16])
        w_v[pl.ds(o, 16)] = jnp.exp(e2_v[pl.ds(o, 16)] - cv)

    pltpu.sync_copy(w_v, w_hbm.at[pl.ds(base, _CHUNK)])


@functools.partial(jax.jit, static_argnames=("interpret",))
def _run(edge_feats, segment_ids, W, b, interpret=False):
    seg3 = segment_ids.astype(jnp.int32).reshape(NB, 1, BE)
    wt = W.reshape(1, D)

    e2_flat, h, c = pl.pallas_call(
        _pass1_body,
        grid=(NB,),
        in_specs=[
            pl.BlockSpec((1, 1, BE), lambda i: (i, 0, 0)),
            pl.BlockSpec((BE, D), lambda i: (i, 0)),
            pl.BlockSpec((1, D), lambda i: (0, 0)),
            pl.BlockSpec(memory_space=pltpu.SMEM),
        ],
        out_specs=[
            pl.BlockSpec((E,), lambda i: (0,)),
            pl.BlockSpec((G, D), lambda i: (0, 0)),
            pl.BlockSpec((G, 1), lambda i: (0, 0)),
        ],
        out_shape=[
            jax.ShapeDtypeStruct((E,), jnp.float32),
            jax.ShapeDtypeStruct((G, D), jnp.float32),
            jax.ShapeDtypeStruct((G, 1), jnp.float32),
        ],
        scratch_shapes=[
            pltpu.VMEM((G, 1), jnp.float32),
            pltpu.VMEM((G, 1), jnp.float32),
            pltpu.VMEM((G, D), jnp.float32),
        ],
        interpret=interpret,
    )(seg3, edge_feats, wt, b)

    seg_flat = segment_ids.astype(jnp.int32)
    c_pad = jnp.concatenate([c.reshape(G), jnp.zeros((128 - G,), jnp.float32)])

    sc_kernel = functools.partial(
        pl.kernel,
        mesh=plsc.VectorSubcoreMesh(core_axis_name="c", subcore_axis_name="s"),
        compiler_params=pltpu.CompilerParams(needs_layout_passes=False),
        out_type=jax.ShapeDtypeStruct((E,), jnp.float32),
        scratch_types=[
            pltpu.VMEM((_CHUNK,), jnp.int32),
            pltpu.VMEM((_CHUNK,), jnp.float32),
            pltpu.VMEM((_CHUNK,), jnp.float32),
            pltpu.VMEM((128,), jnp.float32),
        ],
    )(_pass2_sc)
    w_flat = sc_kernel(seg_flat, e2_flat, c_pad)

    return h, w_flat.reshape(E, 1)


def kernel(edge_feats, segment_ids, W, b, num_graphs):
    del num_graphs
    return _run(edge_feats, segment_ids, W, b)
